# baseline (speedup 1.0000x reference)
import functools

import jax
import jax.numpy as jnp
from jax import lax
from jax.experimental import pallas as pl
from jax.experimental.pallas import tpu as pltpu
from jax.experimental.pallas import tpu_sc as plsc

_N = 10000
_D = 128
_NC = 2
_NS = 16
_NW = _NC * _NS
_SUB = 64
_NP = 10112
_RPS = _NP // _NS


def _strip_chunks():
    off = 0
    while off < _RPS:
        sz = min(_SUB, _RPS - off)
        yield off, sz
        off += sz


def _mesh():
    return plsc.VectorSubcoreMesh(core_axis_name="c", subcore_axis_name="s",
                                  num_cores=_NC, num_subcores=_NS)


def _sc_counts(e_pad):
    csub = 128
    epw = e_pad // _NW
    nb = epw // csub
    assert nb % 2 == 0

    @functools.partial(
        pl.kernel, mesh=_mesh(),
        out_type=[jax.ShapeDtypeStruct((_NC, _NP, _D), jnp.float32)],
        scratch_types=[
            pltpu.VMEM((csub,), jnp.int32),
            pltpu.VMEM((csub,), jnp.int32),
            pltpu.VMEM((csub, _D), jnp.float32),
            pltpu.VMEM_SHARED((_NP, _D), jnp.float32),
        ] + [pltpu.SemaphoreType.DMA] * 4)
    def k(dst_hbm, cnt_out, dstv0, dstv1, onesv, cnt_sp, si0, si1, ss0, ss1):
        dstv = (dstv0, dstv1)
        sem_i = (si0, si1)
        sem_s = (ss0, ss1)
        cid = lax.axis_index("c")
        sid = lax.axis_index("s")
        wid = sid * _NC + cid
        r_base = sid * _RPS

        def zrow(r, _):
            for c in range(_D // 16):
                onesv[r, pl.ds(c * 16, 16)] = jnp.zeros((16,), jnp.float32)
            return 0
        lax.fori_loop(0, csub, zrow, 0)
        for off, sz in _strip_chunks():
            pltpu.sync_copy(onesv.at[pl.ds(0, sz)],
                            cnt_sp.at[pl.ds(r_base + off, sz)])

        def orow(r, _):
            for c in range(_D // 16):
                onesv[r, pl.ds(c * 16, 16)] = jnp.ones((16,), jnp.float32)
            return 0
        lax.fori_loop(0, csub, orow, 0)
        plsc.subcore_barrier()

        e_base = wid * epw

        def issue_dst(j, b):
            pltpu.async_copy(dst_hbm.at[pl.ds(e_base + j * csub, csub)],
                             dstv[b], sem_i[b])

        def wait_dst(b):
            pltpu.make_async_copy(dst_hbm.at[pl.ds(0, csub)], dstv[b],
                                  sem_i[b]).wait()

        def issue_scatter(b):
            pltpu.async_copy(onesv, cnt_sp.at[dstv[b]], sem_s[b], add=True)

        def wait_scatter(b):
            pltpu.make_async_copy(onesv, cnt_sp.at[dstv[b]], sem_s[b]).wait()

        issue_dst(0, 0)
        issue_dst(1, 1)

        def pair(i2, _):
            j0 = i2 * 2
            for b in (0, 1):
                wait_dst(b)
                issue_scatter(b)
            for b in (0, 1):
                wait_scatter(b)
                issue_dst(j0 + b + 2, b)
            return 0
        lax.fori_loop(0, nb // 2 - 1, pair, 0)
        for b in (0, 1):
            wait_dst(b)
            issue_scatter(b)
        wait_scatter(0)
        wait_scatter(1)
        plsc.subcore_barrier()

        for off, sz in _strip_chunks():
            r0 = r_base + off
            pltpu.sync_copy(cnt_sp.at[pl.ds(r0, sz)],
                            cnt_out.at[cid, pl.ds(r0, sz)])

    return k


def _sc_aggregate(e_pad):
    epw = e_pad // _NW
    nb = epw // _SUB
    assert nb % 2 == 0

    @functools.partial(
        pl.kernel, mesh=_mesh(),
        out_type=[jax.ShapeDtypeStruct((_NC, _NP, _D), jnp.float32)],
        scratch_types=[
            pltpu.VMEM((_SUB,), jnp.int32),
            pltpu.VMEM((_SUB,), jnp.int32),
            pltpu.VMEM((_SUB,), jnp.int32),
            pltpu.VMEM((_SUB,), jnp.int32),
            pltpu.VMEM((_SUB, _D), jnp.float32),
            pltpu.VMEM((_SUB, _D), jnp.float32),
            pltpu.VMEM((_SUB, _D), jnp.float32),
            pltpu.VMEM((_SUB, _D), jnp.float32),
            pltpu.VMEM_SHARED((_NP, _D), jnp.float32),
        ] + [pltpu.SemaphoreType.DMA] * 8)
    def k(h_hbm, eh_hbm, src_hbm, dst_hbm, acc_out, srcv0, srcv1, dstv0,
          dstv1, ehv0, ehv1, rowsv0, rowsv1, acc_sp, si0, si1, se0, se1,
          sg0, sg1, ss0, ss1):
        srcv = (srcv0, srcv1)
        dstv = (dstv0, dstv1)
        ehv = (ehv0, ehv1)
        rowsv = (rowsv0, rowsv1)
        sem_i = (si0, si1)
        sem_e = (se0, se1)
        sem_g = (sg0, sg1)
        sem_s = (ss0, ss1)

        cid = lax.axis_index("c")
        sid = lax.axis_index("s")
        wid = sid * _NC + cid
        r_base = sid * _RPS
        e_base = wid * epw

        def zrow(r, _):
            for c in range(_D // 16):
                rowsv0[r, pl.ds(c * 16, 16)] = jnp.zeros((16,), jnp.float32)
            return 0
        lax.fori_loop(0, _SUB, zrow, 0)
        for off, sz in _strip_chunks():
            pltpu.sync_copy(rowsv0.at[pl.ds(0, sz)],
                            acc_sp.at[pl.ds(r_base + off, sz)])
        plsc.subcore_barrier()

        def issue_idx_eh(j, b):
            b0 = e_base + j * _SUB
            pltpu.async_copy(src_hbm.at[pl.ds(b0, _SUB)], srcv[b], sem_i[b])
            pltpu.async_copy(dst_hbm.at[pl.ds(b0, _SUB)], dstv[b], sem_i[b])
            pltpu.async_copy(eh_hbm.at[pl.ds(b0, _SUB)], ehv[b], sem_e[b])

        def wait_idx(b):
            pltpu.make_async_copy(src_hbm.at[pl.ds(0, _SUB)], srcv[b],
                                  sem_i[b]).wait()
            pltpu.make_async_copy(dst_hbm.at[pl.ds(0, _SUB)], dstv[b],
                                  sem_i[b]).wait()

        def wait_eh(b):
            pltpu.make_async_copy(eh_hbm.at[pl.ds(0, _SUB)], ehv[b],
                                  sem_e[b]).wait()

        def issue_gather(b):
            pltpu.async_copy(h_hbm.at[srcv[b]], rowsv[b], sem_g[b])

        def wait_gather(b):
            pltpu.make_async_copy(h_hbm.at[srcv[b]], rowsv[b],
                                  sem_g[b]).wait()

        def issue_scatter(b):
            pltpu.async_copy(rowsv[b], acc_sp.at[dstv[b]], sem_s[b],
                             add=True)

        def wait_scatter(b):
            pltpu.make_async_copy(rowsv[b], acc_sp.at[dstv[b]],
                                  sem_s[b]).wait()

        def multiply(b):
            def mgrp(g, _):
                r0 = g * 8
                for rr in range(8):
                    for c in range(_D // 16):
                        sl = pl.ds(c * 16, 16)
                        rowsv[b][r0 + rr, sl] = (rowsv[b][r0 + rr, sl]
                                                 * ehv[b][r0 + rr, sl])
                return 0
            lax.fori_loop(0, _SUB // 8, mgrp, 0)

        issue_idx_eh(0, 0)
        issue_idx_eh(1, 1)
        wait_idx(0)
        issue_gather(0)
        wait_idx(1)
        issue_gather(1)

        def pair(i2, _):
            j0 = i2 * 2
            for b in (0, 1):
                wait_gather(b)
                wait_eh(b)
                multiply(b)
                issue_scatter(b)
                issue_idx_eh(j0 + b + 2, b)
            for b in (0, 1):
                wait_idx(b)
                wait_scatter(b)
                issue_gather(b)
            return 0
        lax.fori_loop(0, nb // 2 - 1, pair, 0)

        for b in (0, 1):
            wait_gather(b)
            wait_eh(b)
            multiply(b)
            issue_scatter(b)
        wait_scatter(0)
        wait_scatter(1)
        plsc.subcore_barrier()

        for off, sz in _strip_chunks():
            r0 = r_base + off
            pltpu.sync_copy(acc_sp.at[pl.ds(r0, sz)],
                            acc_out.at[cid, pl.ds(r0, sz)])

    return k


def _eh_project(eap, WeT1, be1, WeT2, be2):
    e_pad, de = eap.shape
    blk = 2048

    def body(ea_ref, w1_ref, b1_ref, w2_ref, b2_ref, o1_ref, o2_ref):
        ea = ea_ref[...]
        o1_ref[...] = jnp.dot(ea, w1_ref[...],
                              preferred_element_type=jnp.float32) + b1_ref[...]
        o2_ref[...] = jnp.dot(ea, w2_ref[...],
                              preferred_element_type=jnp.float32) + b2_ref[...]

    return pl.pallas_call(
        body,
        grid=(e_pad // blk,),
        in_specs=[
            pl.BlockSpec((blk, de), lambda i: (i, 0)),
            pl.BlockSpec((de, _D), lambda i: (0, 0)),
            pl.BlockSpec((1, _D), lambda i: (0, 0)),
            pl.BlockSpec((de, _D), lambda i: (0, 0)),
            pl.BlockSpec((1, _D), lambda i: (0, 0)),
        ],
        out_specs=[pl.BlockSpec((blk, _D), lambda i: (i, 0))] * 2,
        out_shape=[jax.ShapeDtypeStruct((e_pad, _D), jnp.float32)] * 2,
    )(eap, WeT1, be1.reshape(1, _D), WeT2, be2.reshape(1, _D))


def _dense_stage(acc, cnt, h, WjT, bj, WiT, g, b, WdT=None, bd=None):
    blk = 400
    final = WdT is not None

    def body(a0_ref, a1_ref, c0_ref, c1_ref, h_ref, wj_ref, bj_ref, wi_ref,
             g_ref, b_ref, *rest):
        if final:
            wd_ref, bd_ref, o_ref = rest
        else:
            (o_ref,) = rest
        s = a0_ref[0] + a1_ref[0]
        c = c0_ref[0][:, :1] + c1_ref[0][:, :1]
        mean = s / jnp.maximum(c, 1.0)
        z = (jnp.dot(mean, wj_ref[...], preferred_element_type=jnp.float32)
             + bj_ref[...]
             + jnp.dot(h_ref[...], wi_ref[...],
                       preferred_element_type=jnp.float32))
        mu = jnp.mean(z, axis=-1, keepdims=True)
        var = jnp.mean((z - mu) ** 2, axis=-1, keepdims=True)
        hn = (z - mu) * lax.rsqrt(var + 1e-5) * g_ref[...] + b_ref[...]
        hr = jnp.maximum(hn, 0.0)
        if final:
            o_ref[...] = jnp.dot(hr, wd_ref[...],
                                 preferred_element_type=jnp.float32) + bd_ref[...]
        else:
            o_ref[...] = hr

    in_specs = [
        pl.BlockSpec((1, blk, _D), lambda i: (0, i, 0)),
        pl.BlockSpec((1, blk, _D), lambda i: (1, i, 0)),
        pl.BlockSpec((1, blk, _D), lambda i: (0, i, 0)),
        pl.BlockSpec((1, blk, _D), lambda i: (1, i, 0)),
        pl.BlockSpec((blk, _D), lambda i: (i, 0)),
        pl.BlockSpec((_D, _D), lambda i: (0, 0)),
        pl.BlockSpec((1, _D), lambda i: (0, 0)),
        pl.BlockSpec((_D, _D), lambda i: (0, 0)),
        pl.BlockSpec((1, _D), lambda i: (0, 0)),
        pl.BlockSpec((1, _D), lambda i: (0, 0)),
    ]
    args = [acc, acc, cnt, cnt, h, WjT, bj.reshape(1, _D), WiT,
            g.reshape(1, _D), b.reshape(1, _D)]
    if final:
        in_specs += [pl.BlockSpec((_D, 1), lambda i: (0, 0)),
                     pl.BlockSpec((1, 1), lambda i: (0, 0))]
        args += [WdT, bd.reshape(1, 1)]
        out_spec = pl.BlockSpec((blk, 1), lambda i: (i, 0))
        out_shape = jax.ShapeDtypeStruct((_N, 1), jnp.float32)
    else:
        out_spec = pl.BlockSpec((blk, _D), lambda i: (i, 0))
        out_shape = jax.ShapeDtypeStruct((_N, _D), jnp.float32)

    return pl.pallas_call(
        body,
        grid=(_N // blk,),
        in_specs=in_specs,
        out_specs=out_spec,
        out_shape=out_shape,
    )(*args)


def kernel(x, edge_attr, edge_index, Wi1, Wj1, bj1, We1, be1, g1, b1,
           Wi2, Wj2, bj2, We2, be2, g2, b2, Wd, bd):
    e = edge_attr.shape[0]
    de = edge_attr.shape[1]
    gran = _NW * 256
    e_pad = ((e + gran - 1) // gran) * gran
    pad = e_pad - e

    src = jnp.concatenate([edge_index[0], jnp.zeros((pad,), jnp.int32)])
    dst = jnp.concatenate([edge_index[1], jnp.full((pad,), _N, jnp.int32)])
    eap = jnp.concatenate([edge_attr, jnp.zeros((pad, de), jnp.float32)])

    eh1, eh2 = _eh_project(eap, We1.T, be1, We2.T, be2)
    (cnt,) = _sc_counts(e_pad)(dst)

    (acc1,) = _sc_aggregate(e_pad)(x, eh1, src, dst)
    h1 = _dense_stage(acc1, cnt, x, Wj1.T, bj1, Wi1.T, g1, b1)

    (acc2,) = _sc_aggregate(e_pad)(h1, eh2, src, dst)
    out = _dense_stage(acc2, cnt, h1, Wj2.T, bj2, Wi2.T, g2, b2, Wd.T, bd)
    return out


# speedup vs baseline: 1.0048x; 1.0048x over previous
import functools

import jax
import jax.numpy as jnp
from jax import lax
from jax.experimental import pallas as pl
from jax.experimental.pallas import tpu as pltpu
from jax.experimental.pallas import tpu_sc as plsc

_N = 10000
_D = 128
_NC = 2
_NS = 16
_NW = _NC * _NS
_SUB = 64
_NP = 10112
_RPS = _NP // _NS


def _strip_chunks():
    off = 0
    while off < _RPS:
        sz = min(_SUB, _RPS - off)
        yield off, sz
        off += sz


def _mesh():
    return plsc.VectorSubcoreMesh(core_axis_name="c", subcore_axis_name="s",
                                  num_cores=_NC, num_subcores=_NS)


def _sc_counts(e_pad):
    csub = 128
    epw = e_pad // _NW
    nb = epw // csub
    assert nb % 2 == 0

    @functools.partial(
        pl.kernel, mesh=_mesh(),
        out_type=[jax.ShapeDtypeStruct((_NC, _NP, _D), jnp.float32)],
        scratch_types=[
            pltpu.VMEM((csub,), jnp.int32),
            pltpu.VMEM((csub,), jnp.int32),
            pltpu.VMEM((csub, _D), jnp.float32),
            pltpu.VMEM_SHARED((_NP, _D), jnp.float32),
        ] + [pltpu.SemaphoreType.DMA] * 4)
    def k(dst_hbm, cnt_out, dstv0, dstv1, onesv, cnt_sp, si0, si1, ss0, ss1):
        dstv = (dstv0, dstv1)
        sem_i = (si0, si1)
        sem_s = (ss0, ss1)
        cid = lax.axis_index("c")
        sid = lax.axis_index("s")
        wid = sid * _NC + cid
        r_base = sid * _RPS

        def zrow(r, _):
            for c in range(_D // 16):
                onesv[r, pl.ds(c * 16, 16)] = jnp.zeros((16,), jnp.float32)
            return 0
        lax.fori_loop(0, csub, zrow, 0)
        for off, sz in _strip_chunks():
            pltpu.sync_copy(onesv.at[pl.ds(0, sz)],
                            cnt_sp.at[pl.ds(r_base + off, sz)])

        def orow(r, _):
            for c in range(_D // 16):
                onesv[r, pl.ds(c * 16, 16)] = jnp.ones((16,), jnp.float32)
            return 0
        lax.fori_loop(0, csub, orow, 0)
        plsc.subcore_barrier()

        e_base = wid * epw

        def issue_dst(j, b):
            pltpu.async_copy(dst_hbm.at[pl.ds(e_base + j * csub, csub)],
                             dstv[b], sem_i[b])

        def wait_dst(b):
            pltpu.make_async_copy(dst_hbm.at[pl.ds(0, csub)], dstv[b],
                                  sem_i[b]).wait()

        def issue_scatter(b):
            pltpu.async_copy(onesv, cnt_sp.at[dstv[b]], sem_s[b], add=True)

        def wait_scatter(b):
            pltpu.make_async_copy(onesv, cnt_sp.at[dstv[b]], sem_s[b]).wait()

        issue_dst(0, 0)
        issue_dst(1, 1)

        def pair(i2, _):
            j0 = i2 * 2
            for b in (0, 1):
                wait_dst(b)
                issue_scatter(b)
            for b in (0, 1):
                wait_scatter(b)
                issue_dst(j0 + b + 2, b)
            return 0
        lax.fori_loop(0, nb // 2 - 1, pair, 0)
        for b in (0, 1):
            wait_dst(b)
            issue_scatter(b)
        wait_scatter(0)
        wait_scatter(1)
        plsc.subcore_barrier()

        for off, sz in _strip_chunks():
            r0 = r_base + off
            pltpu.sync_copy(cnt_sp.at[pl.ds(r0, sz)],
                            cnt_out.at[cid, pl.ds(r0, sz)])

    return k


def _sc_aggregate(e_pad):
    epw = e_pad // _NW
    nb = epw // _SUB
    assert nb % 2 == 0

    @functools.partial(
        pl.kernel, mesh=_mesh(),
        out_type=[jax.ShapeDtypeStruct((_NC, _NP, _D), jnp.float32)],
        scratch_types=[
            pltpu.VMEM((_SUB,), jnp.int32),
            pltpu.VMEM((_SUB,), jnp.int32),
            pltpu.VMEM((_SUB,), jnp.int32),
            pltpu.VMEM((_SUB,), jnp.int32),
            pltpu.VMEM((_SUB, _D), jnp.float32),
            pltpu.VMEM((_SUB, _D), jnp.float32),
            pltpu.VMEM((_SUB, _D), jnp.float32),
            pltpu.VMEM((_SUB, _D), jnp.float32),
            pltpu.VMEM_SHARED((_NP, _D), jnp.float32),
        ] + [pltpu.SemaphoreType.DMA] * 8)
    def k(h_hbm, eh_hbm, src_hbm, dst_hbm, acc_out, srcv0, srcv1, dstv0,
          dstv1, ehv0, ehv1, rowsv0, rowsv1, acc_sp, si0, si1, se0, se1,
          sg0, sg1, ss0, ss1):
        srcv = (srcv0, srcv1)
        dstv = (dstv0, dstv1)
        ehv = (ehv0, ehv1)
        rowsv = (rowsv0, rowsv1)
        sem_i = (si0, si1)
        sem_e = (se0, se1)
        sem_g = (sg0, sg1)
        sem_s = (ss0, ss1)

        cid = lax.axis_index("c")
        sid = lax.axis_index("s")
        wid = sid * _NC + cid
        r_base = sid * _RPS
        e_base = wid * epw

        def zrow(r, _):
            for c in range(_D // 16):
                rowsv0[r, pl.ds(c * 16, 16)] = jnp.zeros((16,), jnp.float32)
            return 0
        lax.fori_loop(0, _SUB, zrow, 0)
        for off, sz in _strip_chunks():
            pltpu.sync_copy(rowsv0.at[pl.ds(0, sz)],
                            acc_sp.at[pl.ds(r_base + off, sz)])
        plsc.subcore_barrier()

        def issue_idx_eh(j, b):
            b0 = e_base + j * _SUB
            pltpu.async_copy(src_hbm.at[pl.ds(b0, _SUB)], srcv[b], sem_i[b])
            pltpu.async_copy(dst_hbm.at[pl.ds(b0, _SUB)], dstv[b], sem_i[b])
            pltpu.async_copy(eh_hbm.at[pl.ds(b0, _SUB)], ehv[b], sem_e[b])

        def wait_idx(b):
            pltpu.make_async_copy(src_hbm.at[pl.ds(0, _SUB)], srcv[b],
                                  sem_i[b]).wait()
            pltpu.make_async_copy(dst_hbm.at[pl.ds(0, _SUB)], dstv[b],
                                  sem_i[b]).wait()

        def wait_eh(b):
            pltpu.make_async_copy(eh_hbm.at[pl.ds(0, _SUB)], ehv[b],
                                  sem_e[b]).wait()

        def issue_gather(b):
            pltpu.async_copy(h_hbm.at[srcv[b]], rowsv[b], sem_g[b])

        def wait_gather(b):
            pltpu.make_async_copy(h_hbm.at[srcv[b]], rowsv[b],
                                  sem_g[b]).wait()

        def issue_scatter(b):
            pltpu.async_copy(rowsv[b], acc_sp.at[dstv[b]], sem_s[b],
                             add=True)

        def wait_scatter(b):
            pltpu.make_async_copy(rowsv[b], acc_sp.at[dstv[b]],
                                  sem_s[b]).wait()

        def multiply(b):
            def mrow(r, _):
                for c in range(_D // 16):
                    sl = pl.ds(c * 16, 16)
                    rowsv[b][r, sl] = rowsv[b][r, sl] * ehv[b][r, sl]
                return 0
            lax.fori_loop(0, _SUB, mrow, 0)

        issue_idx_eh(0, 0)
        issue_idx_eh(1, 1)
        wait_idx(0)
        issue_gather(0)
        wait_idx(1)
        issue_gather(1)

        def pair(i2, _):
            j0 = i2 * 2
            for b in (0, 1):
                wait_gather(b)
                wait_eh(b)
                multiply(b)
                issue_scatter(b)
                issue_idx_eh(j0 + b + 2, b)
            for b in (0, 1):
                wait_idx(b)
                wait_scatter(b)
                issue_gather(b)
            return 0
        lax.fori_loop(0, nb // 2 - 1, pair, 0)

        for b in (0, 1):
            wait_gather(b)
            wait_eh(b)
            multiply(b)
            issue_scatter(b)
        wait_scatter(0)
        wait_scatter(1)
        plsc.subcore_barrier()

        for off, sz in _strip_chunks():
            r0 = r_base + off
            pltpu.sync_copy(acc_sp.at[pl.ds(r0, sz)],
                            acc_out.at[cid, pl.ds(r0, sz)])

    return k


def _eh_project(eap, WeT1, be1, WeT2, be2):
    e_pad, de = eap.shape
    blk = 2048

    def body(ea_ref, w1_ref, b1_ref, w2_ref, b2_ref, o1_ref, o2_ref):
        ea = ea_ref[...]
        o1_ref[...] = jnp.dot(ea, w1_ref[...],
                              preferred_element_type=jnp.float32) + b1_ref[...]
        o2_ref[...] = jnp.dot(ea, w2_ref[...],
                              preferred_element_type=jnp.float32) + b2_ref[...]

    return pl.pallas_call(
        body,
        grid=(e_pad // blk,),
        in_specs=[
            pl.BlockSpec((blk, de), lambda i: (i, 0)),
            pl.BlockSpec((de, _D), lambda i: (0, 0)),
            pl.BlockSpec((1, _D), lambda i: (0, 0)),
            pl.BlockSpec((de, _D), lambda i: (0, 0)),
            pl.BlockSpec((1, _D), lambda i: (0, 0)),
        ],
        out_specs=[pl.BlockSpec((blk, _D), lambda i: (i, 0))] * 2,
        out_shape=[jax.ShapeDtypeStruct((e_pad, _D), jnp.float32)] * 2,
    )(eap, WeT1, be1.reshape(1, _D), WeT2, be2.reshape(1, _D))


def _dense_stage(acc, cnt, h, WjT, bj, WiT, g, b, WdT=None, bd=None):
    blk = 400
    final = WdT is not None

    def body(a0_ref, a1_ref, c0_ref, c1_ref, h_ref, wj_ref, bj_ref, wi_ref,
             g_ref, b_ref, *rest):
        if final:
            wd_ref, bd_ref, o_ref = rest
        else:
            (o_ref,) = rest
        s = a0_ref[0] + a1_ref[0]
        c = c0_ref[0][:, :1] + c1_ref[0][:, :1]
        mean = s / jnp.maximum(c, 1.0)
        z = (jnp.dot(mean, wj_ref[...], preferred_element_type=jnp.float32)
             + bj_ref[...]
             + jnp.dot(h_ref[...], wi_ref[...],
                       preferred_element_type=jnp.float32))
        mu = jnp.mean(z, axis=-1, keepdims=True)
        var = jnp.mean((z - mu) ** 2, axis=-1, keepdims=True)
        hn = (z - mu) * lax.rsqrt(var + 1e-5) * g_ref[...] + b_ref[...]
        hr = jnp.maximum(hn, 0.0)
        if final:
            o_ref[...] = jnp.dot(hr, wd_ref[...],
                                 preferred_element_type=jnp.float32) + bd_ref[...]
        else:
            o_ref[...] = hr

    in_specs = [
        pl.BlockSpec((1, blk, _D), lambda i: (0, i, 0)),
        pl.BlockSpec((1, blk, _D), lambda i: (1, i, 0)),
        pl.BlockSpec((1, blk, _D), lambda i: (0, i, 0)),
        pl.BlockSpec((1, blk, _D), lambda i: (1, i, 0)),
        pl.BlockSpec((blk, _D), lambda i: (i, 0)),
        pl.BlockSpec((_D, _D), lambda i: (0, 0)),
        pl.BlockSpec((1, _D), lambda i: (0, 0)),
        pl.BlockSpec((_D, _D), lambda i: (0, 0)),
        pl.BlockSpec((1, _D), lambda i: (0, 0)),
        pl.BlockSpec((1, _D), lambda i: (0, 0)),
    ]
    args = [acc, acc, cnt, cnt, h, WjT, bj.reshape(1, _D), WiT,
            g.reshape(1, _D), b.reshape(1, _D)]
    if final:
        in_specs += [pl.BlockSpec((_D, 1), lambda i: (0, 0)),
                     pl.BlockSpec((1, 1), lambda i: (0, 0))]
        args += [WdT, bd.reshape(1, 1)]
        out_spec = pl.BlockSpec((blk, 1), lambda i: (i, 0))
        out_shape = jax.ShapeDtypeStruct((_N, 1), jnp.float32)
    else:
        out_spec = pl.BlockSpec((blk, _D), lambda i: (i, 0))
        out_shape = jax.ShapeDtypeStruct((_N, _D), jnp.float32)

    return pl.pallas_call(
        body,
        grid=(_N // blk,),
        in_specs=in_specs,
        out_specs=out_spec,
        out_shape=out_shape,
    )(*args)


def kernel(x, edge_attr, edge_index, Wi1, Wj1, bj1, We1, be1, g1, b1,
           Wi2, Wj2, bj2, We2, be2, g2, b2, Wd, bd):
    e = edge_attr.shape[0]
    de = edge_attr.shape[1]
    gran = _NW * 256
    e_pad = ((e + gran - 1) // gran) * gran
    pad = e_pad - e

    src = jnp.concatenate([edge_index[0], jnp.zeros((pad,), jnp.int32)])
    dst = jnp.concatenate([edge_index[1], jnp.full((pad,), _N, jnp.int32)])
    eap = jnp.concatenate([edge_attr, jnp.zeros((pad, de), jnp.float32)])

    eh1, eh2 = _eh_project(eap, We1.T, be1, We2.T, be2)
    (cnt,) = _sc_counts(e_pad)(dst)

    (acc1,) = _sc_aggregate(e_pad)(x, eh1, src, dst)
    h1 = _dense_stage(acc1, cnt, x, Wj1.T, bj1, Wi1.T, g1, b1)

    (acc2,) = _sc_aggregate(e_pad)(h1, eh2, src, dst)
    out = _dense_stage(acc2, cnt, h1, Wj2.T, bj2, Wi2.T, g2, b2, Wd.T, bd)
    return out


# speedup vs baseline: 1.2992x; 1.2931x over previous
import functools

import jax
import jax.numpy as jnp
from jax import lax
from jax.experimental import pallas as pl
from jax.experimental.pallas import tpu as pltpu
from jax.experimental.pallas import tpu_sc as plsc

_N = 10000
_D = 128
_NC = 2
_NS = 16
_NW = _NC * _NS
_SUB = 64
_NP = 10112
_RPS = _NP // _NS


def _strip_chunks():
    off = 0
    while off < _RPS:
        sz = min(_SUB, _RPS - off)
        yield off, sz
        off += sz


def _mesh():
    return plsc.VectorSubcoreMesh(core_axis_name="c", subcore_axis_name="s",
                                  num_cores=_NC, num_subcores=_NS)


def _sc_counts(e_pad):
    epw = e_pad // _NW
    nb = epw // _SUB

    @functools.partial(
        pl.kernel, mesh=_mesh(),
        out_type=[jax.ShapeDtypeStruct((_NC, _NP, _D), jnp.float32)],
        scratch_types=[
            pltpu.VMEM((_SUB,), jnp.int32),
            pltpu.VMEM((_SUB, _D), jnp.float32),
            pltpu.VMEM_SHARED((_NP, _D), jnp.float32),
        ])
    def k(dst_hbm, cnt_out, dstv, onesv, cnt_sp):
        cid = lax.axis_index("c")
        sid = lax.axis_index("s")
        wid = sid * _NC + cid
        r_base = sid * _RPS

        def zrow(r, _):
            for c in range(_D // 16):
                onesv[r, pl.ds(c * 16, 16)] = jnp.zeros((16,), jnp.float32)
            return 0
        lax.fori_loop(0, _SUB, zrow, 0)
        for off, sz in _strip_chunks():
            pltpu.sync_copy(onesv.at[pl.ds(0, sz)],
                            cnt_sp.at[pl.ds(r_base + off, sz)])

        def orow(r, _):
            for c in range(_D // 16):
                onesv[r, pl.ds(c * 16, 16)] = jnp.ones((16,), jnp.float32)
            return 0
        lax.fori_loop(0, _SUB, orow, 0)
        plsc.subcore_barrier()

        e_base = wid * epw

        def body(i, _):
            pltpu.sync_copy(dst_hbm.at[pl.ds(e_base + i * _SUB, _SUB)], dstv)
            pltpu.sync_copy(onesv, cnt_sp.at[dstv], add=True)
            return 0
        lax.fori_loop(0, nb, body, 0)
        plsc.subcore_barrier()

        for off, sz in _strip_chunks():
            r0 = r_base + off
            pltpu.sync_copy(cnt_sp.at[pl.ds(r0, sz)],
                            cnt_out.at[cid, pl.ds(r0, sz)])

    return k


def _sc_aggregate(e_pad):
    epw = e_pad // _NW
    nb = epw // _SUB
    assert nb % 2 == 0

    @functools.partial(
        pl.kernel, mesh=_mesh(),
        out_type=[jax.ShapeDtypeStruct((_NC, _NP, _D), jnp.float32)],
        scratch_types=[
            pltpu.VMEM((_SUB,), jnp.int32),
            pltpu.VMEM((_SUB,), jnp.int32),
            pltpu.VMEM((_SUB,), jnp.int32),
            pltpu.VMEM((_SUB,), jnp.int32),
            pltpu.VMEM((_SUB, _D), jnp.float32),
            pltpu.VMEM((_SUB, _D), jnp.float32),
            pltpu.VMEM((_SUB, _D), jnp.float32),
            pltpu.VMEM((_SUB, _D), jnp.float32),
            pltpu.VMEM_SHARED((_NP, _D), jnp.float32),
        ] + [pltpu.SemaphoreType.DMA] * 8)
    def k(h_hbm, eh_hbm, src_hbm, dst_hbm, acc_out, srcv0, srcv1, dstv0,
          dstv1, ehv0, ehv1, rowsv0, rowsv1, acc_sp, si0, si1, se0, se1,
          sg0, sg1, ss0, ss1):
        srcv = (srcv0, srcv1)
        dstv = (dstv0, dstv1)
        ehv = (ehv0, ehv1)
        rowsv = (rowsv0, rowsv1)
        sem_i = (si0, si1)
        sem_e = (se0, se1)
        sem_g = (sg0, sg1)
        sem_s = (ss0, ss1)

        cid = lax.axis_index("c")
        sid = lax.axis_index("s")
        wid = sid * _NC + cid
        r_base = sid * _RPS
        e_base = wid * epw

        def zrow(r, _):
            for c in range(_D // 16):
                rowsv0[r, pl.ds(c * 16, 16)] = jnp.zeros((16,), jnp.float32)
            return 0
        lax.fori_loop(0, _SUB, zrow, 0)
        for off, sz in _strip_chunks():
            pltpu.sync_copy(rowsv0.at[pl.ds(0, sz)],
                            acc_sp.at[pl.ds(r_base + off, sz)])
        plsc.subcore_barrier()

        def issue_idx_eh(j, b):
            b0 = e_base + j * _SUB
            pltpu.async_copy(src_hbm.at[pl.ds(b0, _SUB)], srcv[b], sem_i[b])
            pltpu.async_copy(dst_hbm.at[pl.ds(b0, _SUB)], dstv[b], sem_i[b])
            pltpu.async_copy(eh_hbm.at[pl.ds(b0, _SUB)], ehv[b], sem_e[b])

        def wait_idx(b):
            pltpu.make_async_copy(src_hbm.at[pl.ds(0, _SUB)], srcv[b],
                                  sem_i[b]).wait()
            pltpu.make_async_copy(dst_hbm.at[pl.ds(0, _SUB)], dstv[b],
                                  sem_i[b]).wait()

        def wait_eh(b):
            pltpu.make_async_copy(eh_hbm.at[pl.ds(0, _SUB)], ehv[b],
                                  sem_e[b]).wait()

        def issue_gather(b):
            pltpu.async_copy(h_hbm.at[srcv[b]], rowsv[b], sem_g[b])

        def wait_gather(b):
            pltpu.make_async_copy(h_hbm.at[srcv[b]], rowsv[b],
                                  sem_g[b]).wait()

        def issue_scatter(b):
            pltpu.async_copy(rowsv[b], acc_sp.at[dstv[b]], sem_s[b],
                             add=True)

        def wait_scatter(b):
            pltpu.make_async_copy(rowsv[b], acc_sp.at[dstv[b]],
                                  sem_s[b]).wait()

        def multiply(b):
            def mrow(r, _):
                for c in range(_D // 16):
                    sl = pl.ds(c * 16, 16)
                    rowsv[b][r, sl] = rowsv[b][r, sl] * ehv[b][r, sl]
                return 0
            lax.fori_loop(0, _SUB, mrow, 0)

        issue_idx_eh(0, 0)
        issue_idx_eh(1, 1)
        wait_idx(0)
        issue_gather(0)
        wait_idx(1)
        issue_gather(1)

        def pair(i2, _):
            j0 = i2 * 2
            for b in (0, 1):
                wait_gather(b)
                wait_eh(b)
                multiply(b)
                issue_scatter(b)
                issue_idx_eh(j0 + b + 2, b)
            for b in (0, 1):
                wait_idx(b)
                wait_scatter(b)
                issue_gather(b)
            return 0
        lax.fori_loop(0, nb // 2 - 1, pair, 0)

        for b in (0, 1):
            wait_gather(b)
            wait_eh(b)
            multiply(b)
            issue_scatter(b)
        wait_scatter(0)
        wait_scatter(1)
        plsc.subcore_barrier()

        for off, sz in _strip_chunks():
            r0 = r_base + off
            pltpu.sync_copy(acc_sp.at[pl.ds(r0, sz)],
                            acc_out.at[cid, pl.ds(r0, sz)])

    return k


def _eh_project(eap, WeT1, be1, WeT2, be2):
    e_pad, de = eap.shape
    blk = 2048

    def body(ea_ref, w1_ref, b1_ref, w2_ref, b2_ref, o1_ref, o2_ref):
        ea = ea_ref[...]
        o1_ref[...] = jnp.dot(ea, w1_ref[...],
                              preferred_element_type=jnp.float32) + b1_ref[...]
        o2_ref[...] = jnp.dot(ea, w2_ref[...],
                              preferred_element_type=jnp.float32) + b2_ref[...]

    return pl.pallas_call(
        body,
        grid=(e_pad // blk,),
        in_specs=[
            pl.BlockSpec((blk, de), lambda i: (i, 0)),
            pl.BlockSpec((de, _D), lambda i: (0, 0)),
            pl.BlockSpec((1, _D), lambda i: (0, 0)),
            pl.BlockSpec((de, _D), lambda i: (0, 0)),
            pl.BlockSpec((1, _D), lambda i: (0, 0)),
        ],
        out_specs=[pl.BlockSpec((blk, _D), lambda i: (i, 0))] * 2,
        out_shape=[jax.ShapeDtypeStruct((e_pad, _D), jnp.float32)] * 2,
    )(eap, WeT1, be1.reshape(1, _D), WeT2, be2.reshape(1, _D))


def _dense_stage(acc, cnt, h, WjT, bj, WiT, g, b, WdT=None, bd=None):
    blk = 400
    final = WdT is not None

    def body(a0_ref, a1_ref, c0_ref, c1_ref, h_ref, wj_ref, bj_ref, wi_ref,
             g_ref, b_ref, *rest):
        if final:
            wd_ref, bd_ref, o_ref = rest
        else:
            (o_ref,) = rest
        s = a0_ref[0] + a1_ref[0]
        c = c0_ref[0][:, :1] + c1_ref[0][:, :1]
        mean = s / jnp.maximum(c, 1.0)
        z = (jnp.dot(mean, wj_ref[...], preferred_element_type=jnp.float32)
             + bj_ref[...]
             + jnp.dot(h_ref[...], wi_ref[...],
                       preferred_element_type=jnp.float32))
        mu = jnp.mean(z, axis=-1, keepdims=True)
        var = jnp.mean((z - mu) ** 2, axis=-1, keepdims=True)
        hn = (z - mu) * lax.rsqrt(var + 1e-5) * g_ref[...] + b_ref[...]
        hr = jnp.maximum(hn, 0.0)
        if final:
            o_ref[...] = jnp.dot(hr, wd_ref[...],
                                 preferred_element_type=jnp.float32) + bd_ref[...]
        else:
            o_ref[...] = hr

    in_specs = [
        pl.BlockSpec((1, blk, _D), lambda i: (0, i, 0)),
        pl.BlockSpec((1, blk, _D), lambda i: (1, i, 0)),
        pl.BlockSpec((1, blk, _D), lambda i: (0, i, 0)),
        pl.BlockSpec((1, blk, _D), lambda i: (1, i, 0)),
        pl.BlockSpec((blk, _D), lambda i: (i, 0)),
        pl.BlockSpec((_D, _D), lambda i: (0, 0)),
        pl.BlockSpec((1, _D), lambda i: (0, 0)),
        pl.BlockSpec((_D, _D), lambda i: (0, 0)),
        pl.BlockSpec((1, _D), lambda i: (0, 0)),
        pl.BlockSpec((1, _D), lambda i: (0, 0)),
    ]
    args = [acc, acc, cnt, cnt, h, WjT, bj.reshape(1, _D), WiT,
            g.reshape(1, _D), b.reshape(1, _D)]
    if final:
        in_specs += [pl.BlockSpec((_D, 1), lambda i: (0, 0)),
                     pl.BlockSpec((1, 1), lambda i: (0, 0))]
        args += [WdT, bd.reshape(1, 1)]
        out_spec = pl.BlockSpec((blk, 1), lambda i: (i, 0))
        out_shape = jax.ShapeDtypeStruct((_N, 1), jnp.float32)
    else:
        out_spec = pl.BlockSpec((blk, _D), lambda i: (i, 0))
        out_shape = jax.ShapeDtypeStruct((_N, _D), jnp.float32)

    return pl.pallas_call(
        body,
        grid=(_N // blk,),
        in_specs=in_specs,
        out_specs=out_spec,
        out_shape=out_shape,
    )(*args)


def kernel(x, edge_attr, edge_index, Wi1, Wj1, bj1, We1, be1, g1, b1,
           Wi2, Wj2, bj2, We2, be2, g2, b2, Wd, bd):
    e = edge_attr.shape[0]
    de = edge_attr.shape[1]
    gran = _NW * _SUB * 2
    e_pad = ((e + gran - 1) // gran) * gran
    pad = e_pad - e

    src = jnp.concatenate([edge_index[0], jnp.zeros((pad,), jnp.int32)])
    dst = jnp.concatenate([edge_index[1], jnp.full((pad,), _N, jnp.int32)])
    eap = jnp.concatenate([edge_attr, jnp.zeros((pad, de), jnp.float32)])

    eh1, eh2 = _eh_project(eap, We1.T, be1, We2.T, be2)
    (cnt,) = _sc_counts(e_pad)(dst)

    (acc1,) = _sc_aggregate(e_pad)(x, eh1, src, dst)
    h1 = _dense_stage(acc1, cnt, x, Wj1.T, bj1, Wi1.T, g1, b1)

    (acc2,) = _sc_aggregate(e_pad)(h1, eh2, src, dst)
    out = _dense_stage(acc2, cnt, h1, Wj2.T, bj2, Wi2.T, g2, b2, Wd.T, bd)
    return out


# speedup vs baseline: 1.3552x; 1.0431x over previous
import functools

import jax
import jax.numpy as jnp
from jax import lax
from jax.experimental import pallas as pl
from jax.experimental.pallas import tpu as pltpu
from jax.experimental.pallas import tpu_sc as plsc

_N = 10000
_D = 128
_NC = 2
_NS = 16
_NW = _NC * _NS
_SUB = 64
_NP = 10112
_RPS = _NP // _NS


def _strip_chunks():
    off = 0
    while off < _RPS:
        sz = min(_SUB, _RPS - off)
        yield off, sz
        off += sz


def _mesh():
    return plsc.VectorSubcoreMesh(core_axis_name="c", subcore_axis_name="s",
                                  num_cores=_NC, num_subcores=_NS)


def _sc_counts(e_pad):
    epw = e_pad // _NW
    nb = epw // _SUB

    @functools.partial(
        pl.kernel, mesh=_mesh(),
        out_type=[jax.ShapeDtypeStruct((_NC, _NP, _D), jnp.float32)],
        scratch_types=[
            pltpu.VMEM((_SUB,), jnp.int32),
            pltpu.VMEM((_SUB, _D), jnp.float32),
            pltpu.VMEM_SHARED((_NP, _D), jnp.float32),
        ])
    def k(dst_hbm, cnt_out, dstv, onesv, cnt_sp):
        cid = lax.axis_index("c")
        sid = lax.axis_index("s")
        wid = sid * _NC + cid
        r_base = sid * _RPS

        def zrow(r, _):
            for c in range(_D // 16):
                onesv[r, pl.ds(c * 16, 16)] = jnp.zeros((16,), jnp.float32)
            return 0
        lax.fori_loop(0, _SUB, zrow, 0)
        for off, sz in _strip_chunks():
            pltpu.sync_copy(onesv.at[pl.ds(0, sz)],
                            cnt_sp.at[pl.ds(r_base + off, sz)])

        def orow(r, _):
            for c in range(_D // 16):
                onesv[r, pl.ds(c * 16, 16)] = jnp.ones((16,), jnp.float32)
            return 0
        lax.fori_loop(0, _SUB, orow, 0)
        plsc.subcore_barrier()

        e_base = wid * epw

        def body(i, _):
            pltpu.sync_copy(dst_hbm.at[pl.ds(e_base + i * _SUB, _SUB)], dstv)
            pltpu.sync_copy(onesv, cnt_sp.at[dstv], add=True)
            return 0
        lax.fori_loop(0, nb, body, 0)
        plsc.subcore_barrier()

        for off, sz in _strip_chunks():
            r0 = r_base + off
            pltpu.sync_copy(cnt_sp.at[pl.ds(r0, sz)],
                            cnt_out.at[cid, pl.ds(r0, sz)])

    return k


def _sc_aggregate(e_pad):
    epw = e_pad // _NW
    nb = epw // _SUB
    assert nb % 2 == 0

    de4 = 4
    @functools.partial(
        pl.kernel, mesh=_mesh(),
        out_type=[jax.ShapeDtypeStruct((_NC, _NP, _D), jnp.float32)],
        scratch_types=[
            pltpu.VMEM((_SUB,), jnp.int32),
            pltpu.VMEM((_SUB,), jnp.int32),
            pltpu.VMEM((_SUB,), jnp.int32),
            pltpu.VMEM((_SUB,), jnp.int32),
            pltpu.VMEM((_SUB * de4,), jnp.float32),
            pltpu.VMEM((_SUB * de4,), jnp.float32),
            pltpu.VMEM((_SUB, _D), jnp.float32),
            pltpu.VMEM((_SUB, _D), jnp.float32),
            pltpu.VMEM((8, _D), jnp.float32),
            pltpu.VMEM_SHARED((_NP, _D), jnp.float32),
        ] + [pltpu.SemaphoreType.DMA] * 8)
    def k(h_hbm, ea_hbm, w_hbm, src_hbm, dst_hbm, acc_out, srcv0, srcv1,
          dstv0, dstv1, eav0, eav1, rowsv0, rowsv1, wv, acc_sp, si0, si1,
          se0, se1, sg0, sg1, ss0, ss1):
        srcv = (srcv0, srcv1)
        dstv = (dstv0, dstv1)
        eav = (eav0, eav1)
        rowsv = (rowsv0, rowsv1)
        sem_i = (si0, si1)
        sem_e = (se0, se1)
        sem_g = (sg0, sg1)
        sem_s = (ss0, ss1)

        cid = lax.axis_index("c")
        sid = lax.axis_index("s")
        wid = sid * _NC + cid
        r_base = sid * _RPS
        e_base = wid * epw

        def zrow(r, _):
            for c in range(_D // 16):
                rowsv0[r, pl.ds(c * 16, 16)] = jnp.zeros((16,), jnp.float32)
            return 0
        lax.fori_loop(0, _SUB, zrow, 0)
        for off, sz in _strip_chunks():
            pltpu.sync_copy(rowsv0.at[pl.ds(0, sz)],
                            acc_sp.at[pl.ds(r_base + off, sz)])
        pltpu.sync_copy(w_hbm, wv)
        wch = [[wv[kk, pl.ds(c * 16, 16)] for c in range(_D // 16)]
               for kk in range(de4)]
        bch = [wv[de4, pl.ds(c * 16, 16)] for c in range(_D // 16)]
        plsc.subcore_barrier()

        def issue_idx_eh(j, b):
            b0 = e_base + j * _SUB
            pltpu.async_copy(src_hbm.at[pl.ds(b0, _SUB)], srcv[b], sem_i[b])
            pltpu.async_copy(dst_hbm.at[pl.ds(b0, _SUB)], dstv[b], sem_i[b])
            pltpu.async_copy(ea_hbm.at[pl.ds(b0 * de4, _SUB * de4)], eav[b],
                             sem_e[b])

        def wait_idx(b):
            pltpu.make_async_copy(src_hbm.at[pl.ds(0, _SUB)], srcv[b],
                                  sem_i[b]).wait()
            pltpu.make_async_copy(dst_hbm.at[pl.ds(0, _SUB)], dstv[b],
                                  sem_i[b]).wait()

        def wait_eh(b):
            pltpu.make_async_copy(ea_hbm.at[pl.ds(0, _SUB * de4)], eav[b],
                                  sem_e[b]).wait()

        def issue_gather(b):
            pltpu.async_copy(h_hbm.at[srcv[b]], rowsv[b], sem_g[b])

        def wait_gather(b):
            pltpu.make_async_copy(h_hbm.at[srcv[b]], rowsv[b],
                                  sem_g[b]).wait()

        def issue_scatter(b):
            pltpu.async_copy(rowsv[b], acc_sp.at[dstv[b]], sem_s[b],
                             add=True)

        def wait_scatter(b):
            pltpu.make_async_copy(rowsv[b], acc_sp.at[dstv[b]],
                                  sem_s[b]).wait()

        def multiply(b):
            def mgrp(g, _):
                avec = eav[b][pl.ds(g * 16, 16)]
                for rr in range(4):
                    a = [avec[rr * de4 + kk] for kk in range(de4)]
                    for c in range(_D // 16):
                        sl = pl.ds(c * 16, 16)
                        ehc = (bch[c] + a[0] * wch[0][c] + a[1] * wch[1][c]
                               + a[2] * wch[2][c] + a[3] * wch[3][c])
                        rowsv[b][g * 4 + rr, sl] = (rowsv[b][g * 4 + rr, sl]
                                                    * ehc)
                return 0
            lax.fori_loop(0, _SUB // 4, mgrp, 0)

        issue_idx_eh(0, 0)
        issue_idx_eh(1, 1)
        wait_idx(0)
        issue_gather(0)
        wait_idx(1)
        issue_gather(1)

        def pair(i2, _):
            j0 = i2 * 2
            for b in (0, 1):
                wait_gather(b)
                wait_eh(b)
                multiply(b)
                issue_scatter(b)
                issue_idx_eh(j0 + b + 2, b)
            for b in (0, 1):
                wait_idx(b)
                wait_scatter(b)
                issue_gather(b)
            return 0
        lax.fori_loop(0, nb // 2 - 1, pair, 0)

        for b in (0, 1):
            wait_gather(b)
            wait_eh(b)
            multiply(b)
            issue_scatter(b)
        wait_scatter(0)
        wait_scatter(1)
        plsc.subcore_barrier()

        for off, sz in _strip_chunks():
            r0 = r_base + off
            pltpu.sync_copy(acc_sp.at[pl.ds(r0, sz)],
                            acc_out.at[cid, pl.ds(r0, sz)])

    return k


def _dense_stage(acc, cnt, h, WjT, bj, WiT, g, b, WdT=None, bd=None):
    blk = 400
    final = WdT is not None

    def body(a0_ref, a1_ref, c0_ref, c1_ref, h_ref, wj_ref, bj_ref, wi_ref,
             g_ref, b_ref, *rest):
        if final:
            wd_ref, bd_ref, o_ref = rest
        else:
            (o_ref,) = rest
        s = a0_ref[0] + a1_ref[0]
        c = c0_ref[0][:, :1] + c1_ref[0][:, :1]
        mean = s / jnp.maximum(c, 1.0)
        z = (jnp.dot(mean, wj_ref[...], preferred_element_type=jnp.float32)
             + bj_ref[...]
             + jnp.dot(h_ref[...], wi_ref[...],
                       preferred_element_type=jnp.float32))
        mu = jnp.mean(z, axis=-1, keepdims=True)
        var = jnp.mean((z - mu) ** 2, axis=-1, keepdims=True)
        hn = (z - mu) * lax.rsqrt(var + 1e-5) * g_ref[...] + b_ref[...]
        hr = jnp.maximum(hn, 0.0)
        if final:
            o_ref[...] = jnp.dot(hr, wd_ref[...],
                                 preferred_element_type=jnp.float32) + bd_ref[...]
        else:
            o_ref[...] = hr

    in_specs = [
        pl.BlockSpec((1, blk, _D), lambda i: (0, i, 0)),
        pl.BlockSpec((1, blk, _D), lambda i: (1, i, 0)),
        pl.BlockSpec((1, blk, _D), lambda i: (0, i, 0)),
        pl.BlockSpec((1, blk, _D), lambda i: (1, i, 0)),
        pl.BlockSpec((blk, _D), lambda i: (i, 0)),
        pl.BlockSpec((_D, _D), lambda i: (0, 0)),
        pl.BlockSpec((1, _D), lambda i: (0, 0)),
        pl.BlockSpec((_D, _D), lambda i: (0, 0)),
        pl.BlockSpec((1, _D), lambda i: (0, 0)),
        pl.BlockSpec((1, _D), lambda i: (0, 0)),
    ]
    args = [acc, acc, cnt, cnt, h, WjT, bj.reshape(1, _D), WiT,
            g.reshape(1, _D), b.reshape(1, _D)]
    if final:
        in_specs += [pl.BlockSpec((_D, 1), lambda i: (0, 0)),
                     pl.BlockSpec((1, 1), lambda i: (0, 0))]
        args += [WdT, bd.reshape(1, 1)]
        out_spec = pl.BlockSpec((blk, 1), lambda i: (i, 0))
        out_shape = jax.ShapeDtypeStruct((_N, 1), jnp.float32)
    else:
        out_spec = pl.BlockSpec((blk, _D), lambda i: (i, 0))
        out_shape = jax.ShapeDtypeStruct((_N, _D), jnp.float32)

    return pl.pallas_call(
        body,
        grid=(_N // blk,),
        in_specs=in_specs,
        out_specs=out_spec,
        out_shape=out_shape,
    )(*args)


def kernel(x, edge_attr, edge_index, Wi1, Wj1, bj1, We1, be1, g1, b1,
           Wi2, Wj2, bj2, We2, be2, g2, b2, Wd, bd):
    e = edge_attr.shape[0]
    de = edge_attr.shape[1]
    gran = _NW * _SUB * 2
    e_pad = ((e + gran - 1) // gran) * gran
    pad = e_pad - e

    src = jnp.concatenate([edge_index[0], jnp.zeros((pad,), jnp.int32)])
    dst = jnp.concatenate([edge_index[1], jnp.full((pad,), _N, jnp.int32)])
    eap = jnp.concatenate([edge_attr, jnp.zeros((pad, de), jnp.float32)])
    ea1d = eap.reshape(-1)
    zrow3 = jnp.zeros((3, _D), jnp.float32)
    w1p = jnp.concatenate([We1.T, be1.reshape(1, _D), zrow3])
    w2p = jnp.concatenate([We2.T, be2.reshape(1, _D), zrow3])

    (cnt,) = _sc_counts(e_pad)(dst)

    (acc1,) = _sc_aggregate(e_pad)(x, ea1d, w1p, src, dst)
    h1 = _dense_stage(acc1, cnt, x, Wj1.T, bj1, Wi1.T, g1, b1)

    (acc2,) = _sc_aggregate(e_pad)(h1, ea1d, w2p, src, dst)
    out = _dense_stage(acc2, cnt, h1, Wj2.T, bj2, Wi2.T, g2, b2, Wd.T, bd)
    return out


# speedup vs baseline: 1.4508x; 1.0705x over previous
import functools

import jax
import jax.numpy as jnp
from jax import lax
from jax.experimental import pallas as pl
from jax.experimental.pallas import tpu as pltpu
from jax.experimental.pallas import tpu_sc as plsc

_N = 10000
_D = 128
_NC = 2
_NS = 16
_NW = _NC * _NS
_SUB = 112
_NP = 10112
_RPS = _NP // _NS


def _strip_chunks():
    off = 0
    while off < _RPS:
        sz = min(_SUB, _RPS - off)
        yield off, sz
        off += sz


def _mesh():
    return plsc.VectorSubcoreMesh(core_axis_name="c", subcore_axis_name="s",
                                  num_cores=_NC, num_subcores=_NS)


def _sc_counts(e_pad):
    epw = e_pad // _NW
    nb = epw // _SUB

    @functools.partial(
        pl.kernel, mesh=_mesh(),
        out_type=[jax.ShapeDtypeStruct((_NC, _NP, _D), jnp.float32)],
        scratch_types=[
            pltpu.VMEM((_SUB,), jnp.int32),
            pltpu.VMEM((_SUB, _D), jnp.float32),
            pltpu.VMEM_SHARED((_NP, _D), jnp.float32),
        ])
    def k(dst_hbm, cnt_out, dstv, onesv, cnt_sp):
        cid = lax.axis_index("c")
        sid = lax.axis_index("s")
        wid = sid * _NC + cid
        r_base = sid * _RPS

        def zrow(r, _):
            for c in range(_D // 16):
                onesv[r, pl.ds(c * 16, 16)] = jnp.zeros((16,), jnp.float32)
            return 0
        lax.fori_loop(0, _SUB, zrow, 0)
        for off, sz in _strip_chunks():
            pltpu.sync_copy(onesv.at[pl.ds(0, sz)],
                            cnt_sp.at[pl.ds(r_base + off, sz)])

        def orow(r, _):
            for c in range(_D // 16):
                onesv[r, pl.ds(c * 16, 16)] = jnp.ones((16,), jnp.float32)
            return 0
        lax.fori_loop(0, _SUB, orow, 0)
        plsc.subcore_barrier()

        e_base = wid * epw

        def body(i, _):
            pltpu.sync_copy(dst_hbm.at[pl.ds(e_base + i * _SUB, _SUB)], dstv)
            pltpu.sync_copy(onesv, cnt_sp.at[dstv], add=True)
            return 0
        lax.fori_loop(0, nb, body, 0)
        plsc.subcore_barrier()

        for off, sz in _strip_chunks():
            r0 = r_base + off
            pltpu.sync_copy(cnt_sp.at[pl.ds(r0, sz)],
                            cnt_out.at[cid, pl.ds(r0, sz)])

    return k


def _sc_aggregate(e_pad):
    epw = e_pad // _NW
    nb = epw // _SUB
    assert nb % 2 == 0

    de4 = 4
    @functools.partial(
        pl.kernel, mesh=_mesh(),
        out_type=[jax.ShapeDtypeStruct((_NC, _NP, _D), jnp.float32)],
        scratch_types=[
            pltpu.VMEM((_SUB,), jnp.int32),
            pltpu.VMEM((_SUB,), jnp.int32),
            pltpu.VMEM((_SUB,), jnp.int32),
            pltpu.VMEM((_SUB,), jnp.int32),
            pltpu.VMEM((_SUB * de4,), jnp.float32),
            pltpu.VMEM((_SUB * de4,), jnp.float32),
            pltpu.VMEM((_SUB, _D), jnp.float32),
            pltpu.VMEM((_SUB, _D), jnp.float32),
            pltpu.VMEM((8, _D), jnp.float32),
            pltpu.VMEM_SHARED((_NP, _D), jnp.float32),
        ] + [pltpu.SemaphoreType.DMA] * 8)
    def k(h_hbm, ea_hbm, w_hbm, src_hbm, dst_hbm, acc_out, srcv0, srcv1,
          dstv0, dstv1, eav0, eav1, rowsv0, rowsv1, wv, acc_sp, si0, si1,
          se0, se1, sg0, sg1, ss0, ss1):
        srcv = (srcv0, srcv1)
        dstv = (dstv0, dstv1)
        eav = (eav0, eav1)
        rowsv = (rowsv0, rowsv1)
        sem_i = (si0, si1)
        sem_e = (se0, se1)
        sem_g = (sg0, sg1)
        sem_s = (ss0, ss1)

        cid = lax.axis_index("c")
        sid = lax.axis_index("s")
        wid = sid * _NC + cid
        r_base = sid * _RPS
        e_base = wid * epw

        def zrow(r, _):
            for c in range(_D // 16):
                rowsv0[r, pl.ds(c * 16, 16)] = jnp.zeros((16,), jnp.float32)
            return 0
        lax.fori_loop(0, _SUB, zrow, 0)
        for off, sz in _strip_chunks():
            pltpu.sync_copy(rowsv0.at[pl.ds(0, sz)],
                            acc_sp.at[pl.ds(r_base + off, sz)])
        pltpu.sync_copy(w_hbm, wv)
        wch = [[wv[kk, pl.ds(c * 16, 16)] for c in range(_D // 16)]
               for kk in range(de4)]
        bch = [wv[de4, pl.ds(c * 16, 16)] for c in range(_D // 16)]
        plsc.subcore_barrier()

        def issue_idx_eh(j, b):
            b0 = e_base + j * _SUB
            pltpu.async_copy(src_hbm.at[pl.ds(b0, _SUB)], srcv[b], sem_i[b])
            pltpu.async_copy(dst_hbm.at[pl.ds(b0, _SUB)], dstv[b], sem_i[b])
            pltpu.async_copy(ea_hbm.at[pl.ds(b0 * de4, _SUB * de4)], eav[b],
                             sem_e[b])

        def wait_idx(b):
            pltpu.make_async_copy(src_hbm.at[pl.ds(0, _SUB)], srcv[b],
                                  sem_i[b]).wait()
            pltpu.make_async_copy(dst_hbm.at[pl.ds(0, _SUB)], dstv[b],
                                  sem_i[b]).wait()

        def wait_eh(b):
            pltpu.make_async_copy(ea_hbm.at[pl.ds(0, _SUB * de4)], eav[b],
                                  sem_e[b]).wait()

        def issue_gather(b):
            pltpu.async_copy(h_hbm.at[srcv[b]], rowsv[b], sem_g[b])

        def wait_gather(b):
            pltpu.make_async_copy(h_hbm.at[srcv[b]], rowsv[b],
                                  sem_g[b]).wait()

        def issue_scatter(b):
            pltpu.async_copy(rowsv[b], acc_sp.at[dstv[b]], sem_s[b],
                             add=True)

        def wait_scatter(b):
            pltpu.make_async_copy(rowsv[b], acc_sp.at[dstv[b]],
                                  sem_s[b]).wait()

        def multiply(b):
            def mgrp(g, _):
                avec = eav[b][pl.ds(g * 16, 16)]
                for rr in range(4):
                    a = [avec[rr * de4 + kk] for kk in range(de4)]
                    for c in range(_D // 16):
                        sl = pl.ds(c * 16, 16)
                        ehc = (bch[c] + a[0] * wch[0][c] + a[1] * wch[1][c]
                               + a[2] * wch[2][c] + a[3] * wch[3][c])
                        rowsv[b][g * 4 + rr, sl] = (rowsv[b][g * 4 + rr, sl]
                                                    * ehc)
                return 0
            lax.fori_loop(0, _SUB // 4, mgrp, 0)

        issue_idx_eh(0, 0)
        issue_idx_eh(1, 1)
        wait_idx(0)
        issue_gather(0)
        wait_idx(1)
        issue_gather(1)

        def pair(i2, _):
            j0 = i2 * 2
            for b in (0, 1):
                wait_gather(b)
                wait_eh(b)
                multiply(b)
                issue_scatter(b)
                issue_idx_eh(j0 + b + 2, b)
            for b in (0, 1):
                wait_idx(b)
                wait_scatter(b)
                issue_gather(b)
            return 0
        lax.fori_loop(0, nb // 2 - 1, pair, 0)

        for b in (0, 1):
            wait_gather(b)
            wait_eh(b)
            multiply(b)
            issue_scatter(b)
        wait_scatter(0)
        wait_scatter(1)
        plsc.subcore_barrier()

        for off, sz in _strip_chunks():
            r0 = r_base + off
            pltpu.sync_copy(acc_sp.at[pl.ds(r0, sz)],
                            acc_out.at[cid, pl.ds(r0, sz)])

    return k


def _dense_stage(acc, cnt, h, WjT, bj, WiT, g, b, WdT=None, bd=None):
    blk = 400
    final = WdT is not None

    def body(a0_ref, a1_ref, c0_ref, c1_ref, h_ref, wj_ref, bj_ref, wi_ref,
             g_ref, b_ref, *rest):
        if final:
            wd_ref, bd_ref, o_ref = rest
        else:
            (o_ref,) = rest
        s = a0_ref[0] + a1_ref[0]
        c = c0_ref[0][:, :1] + c1_ref[0][:, :1]
        mean = s / jnp.maximum(c, 1.0)
        z = (jnp.dot(mean, wj_ref[...], preferred_element_type=jnp.float32)
             + bj_ref[...]
             + jnp.dot(h_ref[...], wi_ref[...],
                       preferred_element_type=jnp.float32))
        mu = jnp.mean(z, axis=-1, keepdims=True)
        var = jnp.mean((z - mu) ** 2, axis=-1, keepdims=True)
        hn = (z - mu) * lax.rsqrt(var + 1e-5) * g_ref[...] + b_ref[...]
        hr = jnp.maximum(hn, 0.0)
        if final:
            o_ref[...] = jnp.dot(hr, wd_ref[...],
                                 preferred_element_type=jnp.float32) + bd_ref[...]
        else:
            o_ref[...] = hr

    in_specs = [
        pl.BlockSpec((1, blk, _D), lambda i: (0, i, 0)),
        pl.BlockSpec((1, blk, _D), lambda i: (1, i, 0)),
        pl.BlockSpec((1, blk, _D), lambda i: (0, i, 0)),
        pl.BlockSpec((1, blk, _D), lambda i: (1, i, 0)),
        pl.BlockSpec((blk, _D), lambda i: (i, 0)),
        pl.BlockSpec((_D, _D), lambda i: (0, 0)),
        pl.BlockSpec((1, _D), lambda i: (0, 0)),
        pl.BlockSpec((_D, _D), lambda i: (0, 0)),
        pl.BlockSpec((1, _D), lambda i: (0, 0)),
        pl.BlockSpec((1, _D), lambda i: (0, 0)),
    ]
    args = [acc, acc, cnt, cnt, h, WjT, bj.reshape(1, _D), WiT,
            g.reshape(1, _D), b.reshape(1, _D)]
    if final:
        in_specs += [pl.BlockSpec((_D, 1), lambda i: (0, 0)),
                     pl.BlockSpec((1, 1), lambda i: (0, 0))]
        args += [WdT, bd.reshape(1, 1)]
        out_spec = pl.BlockSpec((blk, 1), lambda i: (i, 0))
        out_shape = jax.ShapeDtypeStruct((_N, 1), jnp.float32)
    else:
        out_spec = pl.BlockSpec((blk, _D), lambda i: (i, 0))
        out_shape = jax.ShapeDtypeStruct((_N, _D), jnp.float32)

    return pl.pallas_call(
        body,
        grid=(_N // blk,),
        in_specs=in_specs,
        out_specs=out_spec,
        out_shape=out_shape,
    )(*args)


def kernel(x, edge_attr, edge_index, Wi1, Wj1, bj1, We1, be1, g1, b1,
           Wi2, Wj2, bj2, We2, be2, g2, b2, Wd, bd):
    e = edge_attr.shape[0]
    de = edge_attr.shape[1]
    gran = _NW * _SUB * 2
    e_pad = ((e + gran - 1) // gran) * gran
    pad = e_pad - e

    src = jnp.concatenate([edge_index[0], jnp.zeros((pad,), jnp.int32)])
    dst = jnp.concatenate([edge_index[1], jnp.full((pad,), _N, jnp.int32)])
    eap = jnp.concatenate([edge_attr, jnp.zeros((pad, de), jnp.float32)])
    ea1d = eap.reshape(-1)
    zrow3 = jnp.zeros((3, _D), jnp.float32)
    w1p = jnp.concatenate([We1.T, be1.reshape(1, _D), zrow3])
    w2p = jnp.concatenate([We2.T, be2.reshape(1, _D), zrow3])

    (cnt,) = _sc_counts(e_pad)(dst)

    (acc1,) = _sc_aggregate(e_pad)(x, ea1d, w1p, src, dst)
    h1 = _dense_stage(acc1, cnt, x, Wj1.T, bj1, Wi1.T, g1, b1)

    (acc2,) = _sc_aggregate(e_pad)(h1, ea1d, w2p, src, dst)
    out = _dense_stage(acc2, cnt, h1, Wj2.T, bj2, Wi2.T, g2, b2, Wd.T, bd)
    return out


# speedup vs baseline: 1.4543x; 1.0024x over previous
import functools

import jax
import jax.numpy as jnp
from jax import lax
from jax.experimental import pallas as pl
from jax.experimental.pallas import tpu as pltpu
from jax.experimental.pallas import tpu_sc as plsc

_N = 10000
_D = 128
_NC = 2
_NS = 16
_NW = _NC * _NS
_SUB = 120
_NP = 10112
_RPS = _NP // _NS


def _strip_chunks():
    off = 0
    while off < _RPS:
        sz = min(_SUB, _RPS - off)
        yield off, sz
        off += sz


def _mesh():
    return plsc.VectorSubcoreMesh(core_axis_name="c", subcore_axis_name="s",
                                  num_cores=_NC, num_subcores=_NS)


def _sc_counts(e_pad):
    epw = e_pad // _NW
    nb = epw // _SUB

    @functools.partial(
        pl.kernel, mesh=_mesh(),
        out_type=[jax.ShapeDtypeStruct((_NC, _NP, _D), jnp.float32)],
        scratch_types=[
            pltpu.VMEM((_SUB,), jnp.int32),
            pltpu.VMEM((_SUB, _D), jnp.float32),
            pltpu.VMEM_SHARED((_NP, _D), jnp.float32),
        ])
    def k(dst_hbm, cnt_out, dstv, onesv, cnt_sp):
        cid = lax.axis_index("c")
        sid = lax.axis_index("s")
        wid = sid * _NC + cid
        r_base = sid * _RPS

        def zrow(r, _):
            for c in range(_D // 16):
                onesv[r, pl.ds(c * 16, 16)] = jnp.zeros((16,), jnp.float32)
            return 0
        lax.fori_loop(0, _SUB, zrow, 0)
        for off, sz in _strip_chunks():
            pltpu.sync_copy(onesv.at[pl.ds(0, sz)],
                            cnt_sp.at[pl.ds(r_base + off, sz)])

        def orow(r, _):
            for c in range(_D // 16):
                onesv[r, pl.ds(c * 16, 16)] = jnp.ones((16,), jnp.float32)
            return 0
        lax.fori_loop(0, _SUB, orow, 0)
        plsc.subcore_barrier()

        e_base = wid * epw

        def body(i, _):
            pltpu.sync_copy(dst_hbm.at[pl.ds(e_base + i * _SUB, _SUB)], dstv)
            pltpu.sync_copy(onesv, cnt_sp.at[dstv], add=True)
            return 0
        lax.fori_loop(0, nb, body, 0)
        plsc.subcore_barrier()

        for off, sz in _strip_chunks():
            r0 = r_base + off
            pltpu.sync_copy(cnt_sp.at[pl.ds(r0, sz)],
                            cnt_out.at[cid, pl.ds(r0, sz)])

    return k


def _sc_aggregate(e_pad):
    epw = e_pad // _NW
    nb = epw // _SUB
    assert nb % 2 == 0

    de4 = 4
    @functools.partial(
        pl.kernel, mesh=_mesh(),
        out_type=[jax.ShapeDtypeStruct((_NC, _NP, _D), jnp.float32)],
        scratch_types=[
            pltpu.VMEM((_SUB,), jnp.int32),
            pltpu.VMEM((_SUB,), jnp.int32),
            pltpu.VMEM((_SUB,), jnp.int32),
            pltpu.VMEM((_SUB,), jnp.int32),
            pltpu.VMEM((_SUB * de4,), jnp.float32),
            pltpu.VMEM((_SUB * de4,), jnp.float32),
            pltpu.VMEM((_SUB, _D), jnp.float32),
            pltpu.VMEM((_SUB, _D), jnp.float32),
            pltpu.VMEM((8, _D), jnp.float32),
            pltpu.VMEM_SHARED((_NP, _D), jnp.float32),
        ] + [pltpu.SemaphoreType.DMA] * 8)
    def k(h_hbm, ea_hbm, w_hbm, src_hbm, dst_hbm, acc_out, srcv0, srcv1,
          dstv0, dstv1, eav0, eav1, rowsv0, rowsv1, wv, acc_sp, si0, si1,
          se0, se1, sg0, sg1, ss0, ss1):
        srcv = (srcv0, srcv1)
        dstv = (dstv0, dstv1)
        eav = (eav0, eav1)
        rowsv = (rowsv0, rowsv1)
        sem_i = (si0, si1)
        sem_e = (se0, se1)
        sem_g = (sg0, sg1)
        sem_s = (ss0, ss1)

        cid = lax.axis_index("c")
        sid = lax.axis_index("s")
        wid = sid * _NC + cid
        r_base = sid * _RPS
        e_base = wid * epw

        def zrow(r, _):
            for c in range(_D // 16):
                rowsv0[r, pl.ds(c * 16, 16)] = jnp.zeros((16,), jnp.float32)
            return 0
        lax.fori_loop(0, _SUB, zrow, 0)
        for off, sz in _strip_chunks():
            pltpu.sync_copy(rowsv0.at[pl.ds(0, sz)],
                            acc_sp.at[pl.ds(r_base + off, sz)])
        pltpu.sync_copy(w_hbm, wv)
        wch = [[wv[kk, pl.ds(c * 16, 16)] for c in range(_D // 16)]
               for kk in range(de4)]
        bch = [wv[de4, pl.ds(c * 16, 16)] for c in range(_D // 16)]
        plsc.subcore_barrier()

        def issue_idx_eh(j, b):
            b0 = e_base + j * _SUB
            pltpu.async_copy(src_hbm.at[pl.ds(b0, _SUB)], srcv[b], sem_i[b])
            pltpu.async_copy(dst_hbm.at[pl.ds(b0, _SUB)], dstv[b], sem_i[b])
            pltpu.async_copy(ea_hbm.at[pl.ds(b0 * de4, _SUB * de4)], eav[b],
                             sem_e[b])

        def wait_idx(b):
            pltpu.make_async_copy(src_hbm.at[pl.ds(0, _SUB)], srcv[b],
                                  sem_i[b]).wait()
            pltpu.make_async_copy(dst_hbm.at[pl.ds(0, _SUB)], dstv[b],
                                  sem_i[b]).wait()

        def wait_eh(b):
            pltpu.make_async_copy(ea_hbm.at[pl.ds(0, _SUB * de4)], eav[b],
                                  sem_e[b]).wait()

        def issue_gather(b):
            pltpu.async_copy(h_hbm.at[srcv[b]], rowsv[b], sem_g[b])

        def wait_gather(b):
            pltpu.make_async_copy(h_hbm.at[srcv[b]], rowsv[b],
                                  sem_g[b]).wait()

        def issue_scatter(b):
            pltpu.async_copy(rowsv[b], acc_sp.at[dstv[b]], sem_s[b],
                             add=True)

        def wait_scatter(b):
            pltpu.make_async_copy(rowsv[b], acc_sp.at[dstv[b]],
                                  sem_s[b]).wait()

        def multiply(b):
            def mgrp(g, _):
                avec = eav[b][pl.ds(g * 16, 16)]
                for rr in range(4):
                    a = [avec[rr * de4 + kk] for kk in range(de4)]
                    for c in range(_D // 16):
                        sl = pl.ds(c * 16, 16)
                        ehc = (bch[c] + a[0] * wch[0][c] + a[1] * wch[1][c]
                               + a[2] * wch[2][c] + a[3] * wch[3][c])
                        rowsv[b][g * 4 + rr, sl] = (rowsv[b][g * 4 + rr, sl]
                                                    * ehc)
                return 0
            lax.fori_loop(0, _SUB // 4, mgrp, 0)

        issue_idx_eh(0, 0)
        issue_idx_eh(1, 1)
        wait_idx(0)
        issue_gather(0)
        wait_idx(1)
        issue_gather(1)

        def pair(i2, _):
            j0 = i2 * 2
            for b in (0, 1):
                wait_gather(b)
                wait_eh(b)
                multiply(b)
                issue_scatter(b)
                issue_idx_eh(j0 + b + 2, b)
            for b in (0, 1):
                wait_idx(b)
                wait_scatter(b)
                issue_gather(b)
            return 0
        lax.fori_loop(0, nb // 2 - 1, pair, 0)

        for b in (0, 1):
            wait_gather(b)
            wait_eh(b)
            multiply(b)
            issue_scatter(b)
        wait_scatter(0)
        wait_scatter(1)
        plsc.subcore_barrier()

        for off, sz in _strip_chunks():
            r0 = r_base + off
            pltpu.sync_copy(acc_sp.at[pl.ds(r0, sz)],
                            acc_out.at[cid, pl.ds(r0, sz)])

    return k


def _dense_stage(acc, cnt, h, WjT, bj, WiT, g, b, WdT=None, bd=None):
    blk = 400
    final = WdT is not None

    def body(a0_ref, a1_ref, c0_ref, c1_ref, h_ref, wj_ref, bj_ref, wi_ref,
             g_ref, b_ref, *rest):
        if final:
            wd_ref, bd_ref, o_ref = rest
        else:
            (o_ref,) = rest
        s = a0_ref[0] + a1_ref[0]
        c = c0_ref[0][:, :1] + c1_ref[0][:, :1]
        mean = s / jnp.maximum(c, 1.0)
        z = (jnp.dot(mean, wj_ref[...], preferred_element_type=jnp.float32)
             + bj_ref[...]
             + jnp.dot(h_ref[...], wi_ref[...],
                       preferred_element_type=jnp.float32))
        mu = jnp.mean(z, axis=-1, keepdims=True)
        var = jnp.mean((z - mu) ** 2, axis=-1, keepdims=True)
        hn = (z - mu) * lax.rsqrt(var + 1e-5) * g_ref[...] + b_ref[...]
        hr = jnp.maximum(hn, 0.0)
        if final:
            o_ref[...] = jnp.dot(hr, wd_ref[...],
                                 preferred_element_type=jnp.float32) + bd_ref[...]
        else:
            o_ref[...] = hr

    in_specs = [
        pl.BlockSpec((1, blk, _D), lambda i: (0, i, 0)),
        pl.BlockSpec((1, blk, _D), lambda i: (1, i, 0)),
        pl.BlockSpec((1, blk, _D), lambda i: (0, i, 0)),
        pl.BlockSpec((1, blk, _D), lambda i: (1, i, 0)),
        pl.BlockSpec((blk, _D), lambda i: (i, 0)),
        pl.BlockSpec((_D, _D), lambda i: (0, 0)),
        pl.BlockSpec((1, _D), lambda i: (0, 0)),
        pl.BlockSpec((_D, _D), lambda i: (0, 0)),
        pl.BlockSpec((1, _D), lambda i: (0, 0)),
        pl.BlockSpec((1, _D), lambda i: (0, 0)),
    ]
    args = [acc, acc, cnt, cnt, h, WjT, bj.reshape(1, _D), WiT,
            g.reshape(1, _D), b.reshape(1, _D)]
    if final:
        in_specs += [pl.BlockSpec((_D, 1), lambda i: (0, 0)),
                     pl.BlockSpec((1, 1), lambda i: (0, 0))]
        args += [WdT, bd.reshape(1, 1)]
        out_spec = pl.BlockSpec((blk, 1), lambda i: (i, 0))
        out_shape = jax.ShapeDtypeStruct((_N, 1), jnp.float32)
    else:
        out_spec = pl.BlockSpec((blk, _D), lambda i: (i, 0))
        out_shape = jax.ShapeDtypeStruct((_N, _D), jnp.float32)

    return pl.pallas_call(
        body,
        grid=(_N // blk,),
        in_specs=in_specs,
        out_specs=out_spec,
        out_shape=out_shape,
    )(*args)


def kernel(x, edge_attr, edge_index, Wi1, Wj1, bj1, We1, be1, g1, b1,
           Wi2, Wj2, bj2, We2, be2, g2, b2, Wd, bd):
    e = edge_attr.shape[0]
    de = edge_attr.shape[1]
    gran = _NW * _SUB * 2
    e_pad = ((e + gran - 1) // gran) * gran
    pad = e_pad - e

    src = jnp.concatenate([edge_index[0], jnp.zeros((pad,), jnp.int32)])
    dst = jnp.concatenate([edge_index[1], jnp.full((pad,), _N, jnp.int32)])
    eap = jnp.concatenate([edge_attr, jnp.zeros((pad, de), jnp.float32)])
    ea1d = eap.reshape(-1)
    zrow3 = jnp.zeros((3, _D), jnp.float32)
    w1p = jnp.concatenate([We1.T, be1.reshape(1, _D), zrow3])
    w2p = jnp.concatenate([We2.T, be2.reshape(1, _D), zrow3])

    (cnt,) = _sc_counts(e_pad)(dst)

    (acc1,) = _sc_aggregate(e_pad)(x, ea1d, w1p, src, dst)
    h1 = _dense_stage(acc1, cnt, x, Wj1.T, bj1, Wi1.T, g1, b1)

    (acc2,) = _sc_aggregate(e_pad)(h1, ea1d, w2p, src, dst)
    out = _dense_stage(acc2, cnt, h1, Wj2.T, bj2, Wi2.T, g2, b2, Wd.T, bd)
    return out
